# baseline (speedup 1.0000x reference)
import jax
import jax.numpy as jnp
from jax.experimental import pallas as pl

B, C_IN, H, W = 8, 768, 32, 32
LATENT, K = 256, 8192
HW = H * W
KB = 1024
NKB = K // KB


def _body(zf_ref, cb_ref, zn_ref, cn_ref, idx_ref, zq_ref):
    zfb = zf_ref[...]
    zn = zn_ref[...]
    run_min = jnp.full((HW, 1), jnp.inf, dtype=jnp.float32)
    run_idx = jnp.zeros((HW, 1), dtype=jnp.int32)
    for j in range(NKB):
        cbj = cb_ref[pl.ds(j * KB, KB), :]
        mm = jnp.dot(zfb, cbj.T, preferred_element_type=jnp.float32)
        cnj = cn_ref[:, pl.ds(j * KB, KB)]
        dist = (zn + cnj) - 2.0 * mm
        m = jnp.min(dist, axis=1, keepdims=True)
        iota = jax.lax.broadcasted_iota(jnp.int32, (HW, KB), 1) + j * KB
        bidx = jnp.min(jnp.where(dist == m, iota, jnp.int32(2**30)),
                       axis=1, keepdims=True)
        upd = m < run_min
        run_idx = jnp.where(upd, bidx, run_idx)
        run_min = jnp.where(upd, m, run_min)
    idx_ref[...] = run_idx
    zq = jnp.zeros((HW, LATENT), dtype=jnp.float32)
    for j in range(NKB):
        iota = jax.lax.broadcasted_iota(jnp.int32, (HW, KB), 1) + j * KB
        oh = (run_idx == iota).astype(jnp.bfloat16)
        cbj = cb_ref[pl.ds(j * KB, KB), :]
        hi = cbj.astype(jnp.bfloat16)
        lo = (cbj - hi.astype(jnp.float32)).astype(jnp.bfloat16)
        zq = zq + jnp.dot(oh, hi, preferred_element_type=jnp.float32)
        zq = zq + jnp.dot(oh, lo, preferred_element_type=jnp.float32)
    zq_ref[...] = zfb + (zq - zfb)


def _vq(zf, codebook, zn, cn):
    return pl.pallas_call(
        _body,
        grid=(B,),
        in_specs=[
            pl.BlockSpec((HW, LATENT), lambda b: (b, 0)),
            pl.BlockSpec((K, LATENT), lambda b: (0, 0)),
            pl.BlockSpec((HW, 1), lambda b: (b, 0)),
            pl.BlockSpec((1, K), lambda b: (0, 0)),
        ],
        out_specs=[
            pl.BlockSpec((HW, 1), lambda b: (b, 0)),
            pl.BlockSpec((HW, LATENT), lambda b: (b, 0)),
        ],
        out_shape=[
            jax.ShapeDtypeStruct((B * HW, 1), jnp.int32),
            jax.ShapeDtypeStruct((B * HW, LATENT), jnp.float32),
        ],
    )(zf, codebook, zn, cn)


def kernel(z, W_proj, b_proj, codebook):
    zp = jnp.einsum('bchw,oc->bohw', z, W_proj) + b_proj[None, :, None, None]
    zt = jnp.transpose(zp, (0, 2, 3, 1))
    zf = zt.reshape(-1, LATENT)
    zn = jnp.sum(zf * zf, axis=1, keepdims=True)
    cn = jnp.sum(codebook * codebook, axis=1)[None, :]
    idx, zq = _vq(zf, codebook, zn, cn)
    min_indices = idx.reshape(-1)
    rep_z_q = jnp.transpose(zq.reshape(B, H, W, LATENT), (0, 3, 1, 2))
    return (rep_z_q, min_indices)


# speedup vs baseline: 1.4910x; 1.4910x over previous
import functools

import jax
import jax.numpy as jnp
from jax import lax
from jax.experimental import pallas as pl
from jax.experimental.pallas import tpu as pltpu
from jax.experimental.pallas import tpu_sc as plsc

B, C_IN, H, W = 8, 768, 32, 32
LATENT, K = 256, 8192
HW = H * W
KB = 1024
NKB = K // KB
NTOK = B * HW


def _argmin_body(zf_ref, cb_ref, zn_ref, cn_ref, idx_ref):
    zfb = zf_ref[...]
    zn = zn_ref[...]
    run_min = jnp.full((HW, 1), jnp.inf, dtype=jnp.float32)
    run_idx = jnp.zeros((HW, 1), dtype=jnp.int32)
    for j in range(NKB):
        cbj = cb_ref[pl.ds(j * KB, KB), :]
        mm = jnp.dot(zfb, cbj.T, preferred_element_type=jnp.float32)
        cnj = cn_ref[:, pl.ds(j * KB, KB)]
        dist = (zn + cnj) - 2.0 * mm
        m = jnp.min(dist, axis=1, keepdims=True)
        iota = jax.lax.broadcasted_iota(jnp.int32, (HW, KB), 1) + j * KB
        bidx = jnp.min(jnp.where(dist == m, iota, jnp.int32(2**30)),
                       axis=1, keepdims=True)
        upd = m < run_min
        run_idx = jnp.where(upd, bidx, run_idx)
        run_min = jnp.where(upd, m, run_min)
    idx_ref[...] = run_idx


def _vq_argmin(zf, codebook, zn, cn):
    return pl.pallas_call(
        _argmin_body,
        grid=(B,),
        in_specs=[
            pl.BlockSpec((HW, LATENT), lambda b: (b, 0)),
            pl.BlockSpec((K, LATENT), lambda b: (0, 0)),
            pl.BlockSpec((HW, 1), lambda b: (b, 0)),
            pl.BlockSpec((1, K), lambda b: (0, 0)),
        ],
        out_specs=pl.BlockSpec((HW, 1), lambda b: (b, 0)),
        out_shape=jax.ShapeDtypeStruct((NTOK, 1), jnp.int32),
    )(zf, codebook, zn, cn)


_SC_INFO = plsc.get_sparse_core_info()
_NW = _SC_INFO.num_cores * _SC_INFO.num_subcores
_BPW = NTOK // _NW


def _sc_gather(table, idx):
    mesh = plsc.VectorSubcoreMesh(core_axis_name="c", subcore_axis_name="s")

    @functools.partial(
        pl.kernel, mesh=mesh,
        out_type=jax.ShapeDtypeStruct((NTOK, LATENT), jnp.float32),
        scratch_types=[
            pltpu.VMEM((_BPW,), jnp.int32),
            pltpu.VMEM((_BPW, LATENT), jnp.float32),
            pltpu.SemaphoreType.DMA,
        ],
    )
    def k(table_hbm, idx_hbm, out_hbm, idx_v, rows_v, sem):
        wid = lax.axis_index("s") * _SC_INFO.num_cores + lax.axis_index("c")
        base = wid * _BPW
        pltpu.sync_copy(idx_hbm.at[pl.ds(base, _BPW)], idx_v)
        pltpu.async_copy(table_hbm.at[idx_v], rows_v, sem).wait()
        pltpu.sync_copy(rows_v, out_hbm.at[pl.ds(base, _BPW)])

    return k(table, idx)


def kernel(z, W_proj, b_proj, codebook):
    zp = jnp.einsum('bchw,oc->bohw', z, W_proj) + b_proj[None, :, None, None]
    zt = jnp.transpose(zp, (0, 2, 3, 1))
    zf = zt.reshape(-1, LATENT)
    zn = jnp.sum(zf * zf, axis=1, keepdims=True)
    cn = jnp.sum(codebook * codebook, axis=1)[None, :]
    idx = _vq_argmin(zf, codebook, zn, cn)
    min_indices = idx.reshape(-1)
    zq = _sc_gather(codebook, min_indices)
    st = zf + (zq - zf)
    rep_z_q = jnp.transpose(st.reshape(B, H, W, LATENT), (0, 3, 1, 2))
    return (rep_z_q, min_indices)
